# baseline (speedup 1.0000x reference)
import jax
import jax.numpy as jnp
from jax import lax
from jax.experimental import pallas as pl
from jax.experimental.pallas import tpu as pltpu
from jax.experimental.pallas import tpu_sc as plsc

USER_NUM = 1000000
ITEM_NUM = 100000
BATCH = 16384
NC = 2
NS = 16
L = 16
NW = NC * NS
BPW = BATCH // NW
CHUNK = 128
NCH = BPW // CHUNK

TH_TILES = 10
TH_SLICE = USER_NUM // TH_TILES
AB_SLICE = ITEM_NUM // 2
BN = 10000


def _irt_body(theta_hbm, a_hbm, b_hbm, c_hbm, uid_hbm, iid_hbm, out_hbm,
              th_s, a_s, b_s, c_s,
              uid_v, iid_v, th_v, a_v, b_v, c_v, out_v, bb0, bb1,
              idx_sem, stage_sem, spm_sem, out_sem, *chunk_sems):
    cid = lax.axis_index("c")
    sid = lax.axis_index("s")
    wid = sid * NC + cid
    base = wid * BPW

    ic0 = pltpu.async_copy(uid_hbm.at[pl.ds(base, BPW)], uid_v, idx_sem)
    ic1 = pltpu.async_copy(iid_hbm.at[pl.ds(base, BPW)], iid_v, idx_sem)

    def _stage(src, dst, off, nchunks):
        bbs = [bb0, bb1]
        pend = [None, None]
        for k in range(nchunks):
            b = k % 2
            if pend[b] is not None:
                pend[b].wait()
            sl = pl.ds(off + k * BN, BN)
            pltpu.async_copy(src.at[sl], bbs[b], stage_sem).wait()
            pend[b] = pltpu.async_copy(bbs[b], dst.at[sl], spm_sem)
        for p in pend:
            if p is not None:
                p.wait()

    @pl.when(sid < TH_TILES)
    def _():
        _stage(theta_hbm, th_s, sid * TH_SLICE, TH_SLICE // BN)

    src = [a_hbm, b_hbm, c_hbm]
    dst = [a_s, b_s, c_s]
    for t in range(3):
        for h in range(2):
            @pl.when(sid == TH_TILES + 2 * t + h)
            def _(t=t, h=h):
                _stage(src[t], dst[t], h * AB_SLICE, AB_SLICE // BN)

    ic0.wait()
    ic1.wait()
    plsc.subcore_barrier()

    copies = []
    for j in range(NCH):
        sl = pl.ds(j * CHUNK, CHUNK)
        sem = chunk_sems[j]
        copies.append((
            pltpu.async_copy(th_s.at[uid_v.at[sl]], th_v.at[sl], sem),
            pltpu.async_copy(a_s.at[iid_v.at[sl]], a_v.at[sl], sem),
            pltpu.async_copy(b_s.at[iid_v.at[sl]], b_v.at[sl], sem),
            pltpu.async_copy(c_s.at[iid_v.at[sl]], c_v.at[sl], sem),
        ))

    outs = []
    for j in range(NCH):
        for c in copies[j]:
            c.wait()
        for i in range(j * (CHUNK // L), (j + 1) * (CHUNK // L)):
            sl = pl.ds(i * L, L)
            th = th_v[sl]
            a = a_v[sl]
            b = b_v[sl]
            c = c_v[sl]
            cs = 1.0 / (1.0 + jnp.exp(-c))
            s = 1.0 / (1.0 + jnp.exp(-1.702 * a * (th - b)))
            out_v[sl] = cs + (1.0 - cs) * s
        outs.append(pltpu.async_copy(
            out_v.at[pl.ds(j * CHUNK, CHUNK)],
            out_hbm.at[pl.ds(base + j * CHUNK, CHUNK)], out_sem))
    for o in outs:
        o.wait()


@jax.jit
def _irt_sc(theta, a_tab, b_tab, c_tab, uid, iid):
    mesh = plsc.VectorSubcoreMesh(core_axis_name="c", subcore_axis_name="s")
    return pl.kernel(
        _irt_body,
        mesh=mesh,
        out_type=jax.ShapeDtypeStruct((BATCH,), jnp.float32),
        scratch_types=[
            pltpu.VMEM_SHARED((USER_NUM,), jnp.float32),
            pltpu.VMEM_SHARED((ITEM_NUM,), jnp.float32),
            pltpu.VMEM_SHARED((ITEM_NUM,), jnp.float32),
            pltpu.VMEM_SHARED((ITEM_NUM,), jnp.float32),
            pltpu.VMEM((BPW,), jnp.int32),
            pltpu.VMEM((BPW,), jnp.int32),
            pltpu.VMEM((BPW,), jnp.float32),
            pltpu.VMEM((BPW,), jnp.float32),
            pltpu.VMEM((BPW,), jnp.float32),
            pltpu.VMEM((BPW,), jnp.float32),
            pltpu.VMEM((BPW,), jnp.float32),
            pltpu.VMEM((BN,), jnp.float32),
            pltpu.VMEM((BN,), jnp.float32),
            pltpu.SemaphoreType.DMA,
            pltpu.SemaphoreType.DMA,
            pltpu.SemaphoreType.DMA,
            pltpu.SemaphoreType.DMA,
        ] + [pltpu.SemaphoreType.DMA] * NCH,
    )(theta, a_tab, b_tab, c_tab, uid, iid)


def kernel(user_id, item_id, theta_w, a_w, b_w, c_w):
    uid = user_id.astype(jnp.int32)
    iid = item_id.astype(jnp.int32)
    return _irt_sc(theta_w.reshape(-1), a_w.reshape(-1), b_w.reshape(-1),
                   c_w.reshape(-1), uid, iid)


# speedup vs baseline: 1.2021x; 1.2021x over previous
import jax
import jax.numpy as jnp
from jax import lax
from jax.experimental import pallas as pl
from jax.experimental.pallas import tpu as pltpu
from jax.experimental.pallas import tpu_sc as plsc

BATCH = 16384
NC = 2
NS = 16
L = 16
NW = NC * NS
BPW = BATCH // NW
CHUNK = 128
NCH = BPW // CHUNK


def _irt_body(theta_hbm, a_hbm, b_hbm, c_hbm, uid_hbm, iid_hbm, out_hbm,
              uid_v, iid_v, th_v, a_v, b_v, c_v, out_v,
              idx_sem, out_sem, *chunk_sems):
    wid = lax.axis_index("s") * NC + lax.axis_index("c")
    base = wid * BPW

    bsl = pl.ds(base, BPW)
    ic0 = pltpu.async_copy(uid_hbm.at[:, bsl], uid_v, idx_sem)
    ic1 = pltpu.async_copy(iid_hbm.at[:, bsl], iid_v, idx_sem)
    ic0.wait()
    ic1.wait()

    copies = []
    for j in range(NCH):
        sl = pl.ds(j * CHUNK, CHUNK)
        sem = chunk_sems[j]
        copies.append((
            pltpu.async_copy(theta_hbm.at[uid_v.at[:, sl]], th_v.at[:, sl], sem),
            pltpu.async_copy(a_hbm.at[iid_v.at[:, sl]], a_v.at[:, sl], sem),
            pltpu.async_copy(b_hbm.at[iid_v.at[:, sl]], b_v.at[:, sl], sem),
            pltpu.async_copy(c_hbm.at[iid_v.at[:, sl]], c_v.at[:, sl], sem),
        ))

    outs = []
    for j in range(NCH):
        for c in copies[j]:
            c.wait()
        for i in range(j * (CHUNK // L), (j + 1) * (CHUNK // L)):
            sl = pl.ds(i * L, L)
            th = th_v[0, sl]
            a = a_v[0, sl]
            b = b_v[0, sl]
            c = c_v[0, sl]
            cs = 1.0 / (1.0 + jnp.exp(-c))
            s = 1.0 / (1.0 + jnp.exp(-1.702 * a * (th - b)))
            out_v[sl] = cs + (1.0 - cs) * s
        outs.append(pltpu.async_copy(
            out_v.at[pl.ds(j * CHUNK, CHUNK)],
            out_hbm.at[pl.ds(base + j * CHUNK, CHUNK)], out_sem))
    for o in outs:
        o.wait()


@jax.jit
def _irt_sc(theta, a_tab, b_tab, c_tab, uid, iid):
    mesh = plsc.VectorSubcoreMesh(core_axis_name="c", subcore_axis_name="s")
    return pl.kernel(
        _irt_body,
        mesh=mesh,
        compiler_params=pltpu.CompilerParams(use_tc_tiling_on_sc=False),
        out_type=jax.ShapeDtypeStruct((BATCH,), jnp.float32),
        scratch_types=[
            pltpu.VMEM((1, BPW), jnp.int32),
            pltpu.VMEM((1, BPW), jnp.int32),
            pltpu.VMEM((1, BPW), jnp.float32),
            pltpu.VMEM((1, BPW), jnp.float32),
            pltpu.VMEM((1, BPW), jnp.float32),
            pltpu.VMEM((1, BPW), jnp.float32),
            pltpu.VMEM((BPW,), jnp.float32),
            pltpu.SemaphoreType.DMA,
            pltpu.SemaphoreType.DMA,
        ] + [pltpu.SemaphoreType.DMA] * NCH,
    )(theta, a_tab, b_tab, c_tab, uid, iid)


def kernel(user_id, item_id, theta_w, a_w, b_w, c_w):
    uid = user_id.astype(jnp.int32)[None, :]
    iid = item_id.astype(jnp.int32)[None, :]
    return _irt_sc(theta_w.T, a_w.T, b_w.T, c_w.T, uid, iid)
